# baseline (speedup 1.0000x reference)
import functools

import jax
import jax.numpy as jnp
from jax import lax
from jax.experimental import pallas as pl
from jax.experimental.pallas import tpu as pltpu, tpu_sc as plsc

NUM_CLASS = 1000000
EMBED_DIM = 32
BATCH = 16384

_info = plsc.get_sparse_core_info()
_NC, _NS = _info.num_cores, _info.num_subcores
_NW = _NC * _NS
_BPW = BATCH // _NW
_G = 16
_NG = _BPW // _G
_RING = 24


@functools.partial(
    pl.kernel,
    mesh=plsc.VectorSubcoreMesh(core_axis_name="c", subcore_axis_name="s"),
    out_type=jax.ShapeDtypeStruct((EMBED_DIM, BATCH), jnp.float32),
    scratch_types=[
        pltpu.VMEM((_BPW,), jnp.int32),
        pltpu.VMEM((_RING, EMBED_DIM, 128), jnp.float32),
        pltpu.VMEM((EMBED_DIM, 128), jnp.float32),
        pltpu.SemaphoreType.DMA,
    ],
    compiler_params=pltpu.CompilerParams(needs_layout_passes=False),
)
def _embed_gather_t(label_hbm, tablet_hbm, outt_hbm, idx_v, slots_v, out_v, sem):
    wid = lax.axis_index("s") * _NC + lax.axis_index("c")
    base = wid * _BPW
    pltpu.sync_copy(label_hbm.at[pl.ds(base, _BPW)], idx_v)

    c_lo = lax.broadcasted_iota(jnp.int32, (16,), 0)
    c_hi = c_lo + 16

    def group_offsets(g):
        jv = idx_v[pl.ds(pl.multiple_of(g * _G, _G), _G)]
        return (jv // 128) * 128, jv % 128

    def fire(off_scalar, b):
        pltpu.async_copy(
            tablet_hbm.at[
                pl.ds(0, EMBED_DIM), pl.ds(pl.multiple_of(off_scalar, 128), 128)
            ],
            slots_v.at[b],
            sem,
        )

    kv0, _ = group_offsets(0)
    kv1, _ = group_offsets(1)
    for m in range(_G):
        fire(kv0[m], m)
    for m in range(_RING - _G):
        fire(kv1[m], _G + m)

    @pl.loop(0, _NG)
    def _(g):
        _, lv = group_offsets(g)
        kvn1, _ = group_offsets((g + 1) % _NG)
        kvn2, _ = group_offsets((g + 2) % _NG)
        for m in range(_G):
            k = g * _G + m
            slot = k % _RING
            pltpu.make_async_copy(
                tablet_hbm.at[pl.ds(0, EMBED_DIM), pl.ds(0, 128)],
                slots_v.at[slot],
                sem,
            ).wait()
            lane = jnp.full((16,), lv[m], jnp.int32)
            col = jnp.full((16,), (g % 8) * _G + m, jnp.int32)
            vals_lo = plsc.load_gather(slots_v.at[slot], [c_lo, lane])
            vals_hi = plsc.load_gather(slots_v.at[slot], [c_hi, lane])

            @pl.when(k + _RING < _BPW)
            def _():
                if m < _G - (_RING - _G):
                    fire(kvn1[m + (_RING - _G)], slot)
                else:
                    fire(kvn2[m - (_G - (_RING - _G))], slot)

            plsc.store_scatter(out_v, [c_lo, col], vals_lo)
            plsc.store_scatter(out_v, [c_hi, col], vals_hi)

        @pl.when(g % 8 == 7)
        def _():
            out_off = pl.multiple_of(base + (g // 8) * 128, 128)
            pltpu.sync_copy(out_v, outt_hbm.at[:, pl.ds(out_off, 128)])


def kernel(label, embed_table):
    outt = _embed_gather_t(label.astype(jnp.int32), embed_table.T)
    return outt.T


# speedup vs baseline: 1.0282x; 1.0282x over previous
import functools

import jax
import jax.numpy as jnp
from jax import lax
from jax.experimental import pallas as pl
from jax.experimental.pallas import tpu as pltpu, tpu_sc as plsc

NUM_CLASS = 1000000
EMBED_DIM = 32
BATCH = 16384

_info = plsc.get_sparse_core_info()
_NC, _NS = _info.num_cores, _info.num_subcores
_NW = _NC * _NS
_BPW = BATCH // _NW
_G = 16
_NG = _BPW // _G
_RING = 16


@functools.partial(
    pl.kernel,
    mesh=plsc.VectorSubcoreMesh(core_axis_name="c", subcore_axis_name="s"),
    out_type=jax.ShapeDtypeStruct((EMBED_DIM, BATCH), jnp.float32),
    scratch_types=[
        pltpu.VMEM((_BPW,), jnp.int32),
        pltpu.VMEM((_RING, EMBED_DIM, 128), jnp.float32),
        pltpu.VMEM((EMBED_DIM, 128), jnp.float32),
        pltpu.SemaphoreType.DMA,
    ],
    compiler_params=pltpu.CompilerParams(needs_layout_passes=False),
)
def _embed_gather_t(label_hbm, tablet_hbm, outt_hbm, idx_v, slots_v, out_v, sem):
    wid = lax.axis_index("s") * _NC + lax.axis_index("c")
    base = wid * _BPW
    pltpu.sync_copy(label_hbm.at[pl.ds(base, _BPW)], idx_v)

    c_lo = lax.broadcasted_iota(jnp.int32, (16,), 0)
    c_hi = c_lo + 16

    def group_offsets(g):
        jv = idx_v[pl.ds(pl.multiple_of(g * _G, _G), _G)]
        return (jv // 128) * 128, jv % 128

    def fire(off_scalar, b):
        pltpu.async_copy(
            tablet_hbm.at[
                pl.ds(0, EMBED_DIM), pl.ds(pl.multiple_of(off_scalar, 128), 128)
            ],
            slots_v.at[b],
            sem,
        )

    kv0, _ = group_offsets(0)
    kv1, _ = group_offsets(1)
    for m in range(_G):
        fire(kv0[m], m)
    for m in range(_RING - _G):
        fire(kv1[m], _G + m)

    @pl.loop(0, _NG)
    def _(g):
        _, lv = group_offsets(g)
        kvn1, _ = group_offsets((g + 1) % _NG)
        kvn2, _ = group_offsets((g + 2) % _NG)
        for m in range(_G):
            k = g * _G + m
            slot = k % _RING
            pltpu.make_async_copy(
                tablet_hbm.at[pl.ds(0, EMBED_DIM), pl.ds(0, 128)],
                slots_v.at[slot],
                sem,
            ).wait()
            lane = jnp.full((16,), lv[m], jnp.int32)
            col = jnp.full((16,), (g % 8) * _G + m, jnp.int32)
            vals_lo = plsc.load_gather(slots_v.at[slot], [c_lo, lane])
            vals_hi = plsc.load_gather(slots_v.at[slot], [c_hi, lane])

            @pl.when(k + _RING < _BPW)
            def _():
                if m < _G - (_RING - _G):
                    fire(kvn1[m + (_RING - _G)], slot)
                else:
                    fire(kvn2[m - (_G - (_RING - _G))], slot)

            plsc.store_scatter(out_v, [c_lo, col], vals_lo)
            plsc.store_scatter(out_v, [c_hi, col], vals_hi)

        @pl.when(g % 8 == 7)
        def _():
            out_off = pl.multiple_of(base + (g // 8) * 128, 128)
            pltpu.sync_copy(out_v, outt_hbm.at[:, pl.ds(out_off, 128)])


def kernel(label, embed_table):
    outt = _embed_gather_t(label.astype(jnp.int32), embed_table.T)
    return outt.T
